# Initial kernel scaffold; baseline (speedup 1.0000x reference)
import jax
import jax.numpy as jnp
from jax.experimental import pallas as pl


def kernel(x, edge_index, W, a):
    raise NotImplementedError("write your pallas kernel here")


# speedup vs baseline: 14.7583x; 14.7583x over previous
import functools

import jax
import jax.numpy as jnp
from jax import lax
from jax.experimental import pallas as pl
from jax.experimental.pallas import tpu as pltpu
from jax.experimental.pallas import tpu_sc as plsc

N = 10000
E = 320000
DIM = 128
C = 16
ALPHA = 0.2

NC = 2
NS = 16
NW = NC * NS
SUB = 128
KSUB = 16
CHUNK = SUB * KSUB
NCH = 5
EPW = CHUNK * NCH
EPAD = EPW * NW
NPAD = N + 16
GROUPS = CHUNK // 16
ROWS_PER_W = EPW // SUB
ZR = 624
WR = 624


_BLK = 1000


def _prep_body(x_ref, w_ref, a_ref, h_ref, s_ref, t_ref):
    h = jnp.dot(x_ref[...], w_ref[...], preferred_element_type=jnp.float32)
    h_ref[...] = h
    a = a_ref[...]
    s_ref[...] = jnp.sum(h * a[0, :C][None, :], axis=1)
    t_ref[...] = jnp.sum(h * a[0, C:][None, :], axis=1)


_prep = pl.pallas_call(
    _prep_body,
    out_shape=[
        jax.ShapeDtypeStruct((N, C), jnp.float32),
        jax.ShapeDtypeStruct((N,), jnp.float32),
        jax.ShapeDtypeStruct((N,), jnp.float32),
    ],
)


_mesh = plsc.VectorSubcoreMesh(core_axis_name="c", subcore_axis_name="s")


@functools.partial(
    pl.kernel,
    out_type=[
        jax.ShapeDtypeStruct((NC, N, C), jnp.float32),
        jax.ShapeDtypeStruct((NC, N), jnp.float32),
    ],
    mesh=_mesh,
    compiler_params=pltpu.CompilerParams(needs_layout_passes=False,
                                         use_tc_tiling_on_sc=False),
    scratch_types=[
        pltpu.VMEM((NPAD,), jnp.float32),
        pltpu.VMEM((NPAD,), jnp.float32),
        pltpu.VMEM((KSUB, SUB), jnp.int32),
        pltpu.VMEM((KSUB, SUB), jnp.int32),
        pltpu.VMEM((CHUNK, C), jnp.float32),
        pltpu.VMEM((CHUNK, C), jnp.float32),
        pltpu.VMEM((CHUNK,), jnp.float32),
        pltpu.VMEM_SHARED((NPAD, C), jnp.float32),
        pltpu.VMEM_SHARED((NPAD,), jnp.float32),
        pltpu.SemaphoreType.DMA,
    ],
)
def _edges(h_hbm, s_hbm, t_hbm, src2d_hbm, dst2d_hbm,
           outp, outr, s_v, t_v, src2d, dst2d, hd, wv, eev,
           acc_sp, racc_sp, sem):
    c = lax.axis_index("c")
    sid = lax.axis_index("s")
    wid = c * NS + sid

    pltpu.sync_copy(s_hbm, s_v)
    pltpu.sync_copy(t_hbm, t_v)

    zero16 = jnp.zeros((16,), jnp.float32)

    def _zrow(j, carry):
        wv[j, :] = zero16
        return carry

    lax.fori_loop(0, 656, _zrow, 0)

    def _zee(j, carry):
        eev[pl.ds(j * 16, 16)] = zero16
        return carry

    lax.fori_loop(0, GROUPS, _zee, 0)

    @pl.when(sid < 15)
    def _():
        pltpu.sync_copy(wv.at[pl.ds(0, ZR)], acc_sp.at[pl.ds(sid * ZR, ZR)])
        pltpu.sync_copy(eev.at[pl.ds(0, 640)],
                        racc_sp.at[pl.ds(sid * 640, 640)])

    @pl.when(sid == 15)
    def _():
        pltpu.sync_copy(wv.at[pl.ds(0, 656)], acc_sp.at[pl.ds(9360, 656)])
        pltpu.sync_copy(eev.at[pl.ds(0, 416)], racc_sp.at[pl.ds(9600, 416)])

    plsc.subcore_barrier()

    for m in range(NCH):
        rowb = wid * ROWS_PER_W + m * KSUB
        pltpu.sync_copy(src2d_hbm.at[pl.ds(rowb, KSUB)], src2d)
        pltpu.sync_copy(dst2d_hbm.at[pl.ds(rowb, KSUB)], dst2d)

        cps = [
            pltpu.async_copy(h_hbm.at[dst2d.at[k]],
                             hd.at[pl.ds(k * SUB, SUB)], sem)
            for k in range(KSUB)
        ]
        for cp in cps:
            cp.wait()

        def _grp(j, carry):
            off = j * 16
            r = j // 8
            q = (j % 8) * 16
            si = src2d[r, pl.ds(q, 16)]
            di = dst2d[r, pl.ds(q, 16)]
            z = plsc.load_gather(s_v, [si]) + plsc.load_gather(t_v, [di])
            ee = jnp.exp(-jnp.maximum(z, ALPHA * z))
            eev[pl.ds(off, 16)] = ee
            for l in range(16):
                wv[off + l, :] = hd[off + l, :] * ee[l]
            return carry

        lax.fori_loop(0, GROUPS, _grp, 0)

        for k in range(KSUB):
            pltpu.sync_copy(wv.at[pl.ds(k * SUB, SUB)],
                            acc_sp.at[src2d.at[k]], add=True)
            pltpu.sync_copy(eev.at[pl.ds(k * SUB, SUB)],
                            racc_sp.at[src2d.at[k]], add=True)

    plsc.subcore_barrier()

    @pl.when(sid < 15)
    def _():
        pltpu.sync_copy(acc_sp.at[pl.ds(sid * WR, WR)], wv.at[pl.ds(0, WR)])
        pltpu.sync_copy(wv.at[pl.ds(0, WR)], outp.at[c, pl.ds(sid * WR, WR)])
        pltpu.sync_copy(racc_sp.at[pl.ds(sid * 640, 640)],
                        eev.at[pl.ds(0, 640)])
        pltpu.sync_copy(eev.at[pl.ds(0, 640)],
                        outr.at[c, pl.ds(sid * 640, 640)])

    @pl.when(sid == 15)
    def _():
        pltpu.sync_copy(acc_sp.at[pl.ds(9360, 640)], wv.at[pl.ds(0, 640)])
        pltpu.sync_copy(wv.at[pl.ds(0, 640)], outp.at[c, pl.ds(9360, 640)])
        pltpu.sync_copy(racc_sp.at[pl.ds(9600, 400)], eev.at[pl.ds(0, 400)])
        pltpu.sync_copy(eev.at[pl.ds(0, 400)], outr.at[c, pl.ds(9600, 400)])


def _combine_body(p0_ref, p1_ref, r0_ref, r1_ref, o_ref):
    num = p0_ref[...] + p1_ref[...]
    den = r0_ref[...] + r1_ref[...]
    hp = num / den[:, None]
    o_ref[...] = jnp.where(hp > 0, hp, jnp.exp(jnp.minimum(hp, 0.0)) - 1.0)


_combine = pl.pallas_call(
    _combine_body,
    out_shape=jax.ShapeDtypeStruct((N, C), jnp.float32),
)


def kernel(x, edge_index, W, a):
    src = edge_index[0]
    dst = edge_index[1]
    h, s, t = _prep(x, W, a)
    pad = EPAD - E
    srcp = jnp.concatenate([src, jnp.full((pad,), N, jnp.int32)])
    dstp = jnp.concatenate([dst, jnp.zeros((pad,), jnp.int32)])
    sp = jnp.concatenate([s, jnp.zeros((NPAD - N,), jnp.float32)])
    tp = jnp.concatenate([t, jnp.zeros((NPAD - N,), jnp.float32)])
    src2d = srcp.reshape(EPAD // SUB, SUB)
    dst2d = dstp.reshape(EPAD // SUB, SUB)
    outp, outr = _edges(h, sp, tp, src2d, dst2d)
    return _combine(outp[0], outp[1], outr[0], outr[1])


# speedup vs baseline: 21.2141x; 1.4374x over previous
import functools

import jax
import jax.numpy as jnp
from jax import lax
from jax.experimental import pallas as pl
from jax.experimental.pallas import tpu as pltpu
from jax.experimental.pallas import tpu_sc as plsc

N = 10000
E = 320000
DIM = 128
C = 16
ALPHA = 0.2

NC = 2
NS = 16
NW = NC * NS
SUB = 128
KSUB = 8
CHUNK = SUB * KSUB
NCH = 10
EPW = CHUNK * NCH
EPAD = EPW * NW
NPAD = N + 16
GROUPS = CHUNK // 16
ROWS_PER_W = EPW // SUB
ROWS_PER_CH = KSUB
ZR = 624
WR = 624


def _prep_body(x_ref, w_ref, a_ref, h_ref, s_ref, t_ref):
    h = jnp.dot(x_ref[...], w_ref[...], preferred_element_type=jnp.float32)
    h_ref[...] = h
    a = a_ref[...]
    s_ref[...] = jnp.concatenate(
        [jnp.sum(h * a[0, :C][None, :], axis=1),
         jnp.zeros((NPAD - N,), jnp.float32)])
    t_ref[...] = jnp.concatenate(
        [jnp.sum(h * a[0, C:][None, :], axis=1),
         jnp.zeros((NPAD - N,), jnp.float32)])


_prep = pl.pallas_call(
    _prep_body,
    out_shape=[
        jax.ShapeDtypeStruct((N, C), jnp.float32),
        jax.ShapeDtypeStruct((NPAD,), jnp.float32),
        jax.ShapeDtypeStruct((NPAD,), jnp.float32),
    ],
)


_mesh = plsc.VectorSubcoreMesh(core_axis_name="c", subcore_axis_name="s")


@functools.partial(
    pl.kernel,
    out_type=[
        jax.ShapeDtypeStruct((NC, N, C), jnp.float32),
        jax.ShapeDtypeStruct((NC, NS, N), jnp.float32),
    ],
    mesh=_mesh,
    compiler_params=pltpu.CompilerParams(needs_layout_passes=False,
                                         use_tc_tiling_on_sc=False),
    scratch_types=[
        pltpu.VMEM((NPAD,), jnp.float32),
        pltpu.VMEM((NPAD,), jnp.float32),
        pltpu.VMEM((NPAD,), jnp.float32),
        pltpu.VMEM((3, ROWS_PER_CH, SUB), jnp.int32),
        pltpu.VMEM((2, ROWS_PER_CH, SUB), jnp.int32),
        pltpu.VMEM((2, CHUNK, C), jnp.float32),
        pltpu.VMEM((2, CHUNK, C), jnp.float32),
        pltpu.VMEM_SHARED((NPAD, C), jnp.float32),
        pltpu.SemaphoreType.DMA,
        pltpu.SemaphoreType.DMA,
    ],
)
def _edges(h_hbm, s_hbm, t_hbm, src2d_hbm, dst2d_hbm,
           outp, outr, s_v, t_v, rsum_v, srci, dsti, hd, wv,
           acc_sp, gsem, ssem):
    c = lax.axis_index("c")
    sid = lax.axis_index("s")
    wid = c * NS + sid

    pltpu.sync_copy(s_hbm, s_v)
    pltpu.sync_copy(t_hbm, t_v)

    zero16 = jnp.zeros((16,), jnp.float32)

    def _zrow(j, carry):
        wv[0, j, :] = zero16
        rsum_v[pl.ds(j * 16, 16)] = zero16
        return carry

    lax.fori_loop(0, NPAD // 16, _zrow, 0)

    @pl.when(sid < 15)
    def _():
        pltpu.sync_copy(wv.at[0, pl.ds(0, ZR)],
                        acc_sp.at[pl.ds(sid * ZR, ZR)])

    @pl.when(sid == 15)
    def _():
        pltpu.sync_copy(wv.at[0, pl.ds(0, 656)], acc_sp.at[pl.ds(9360, 656)])

    plsc.subcore_barrier()

    def _stage(m):
        rowb = wid * ROWS_PER_W + m * ROWS_PER_CH
        pltpu.sync_copy(src2d_hbm.at[pl.ds(rowb, ROWS_PER_CH)],
                        srci.at[m % 3])
        pltpu.sync_copy(dst2d_hbm.at[pl.ds(rowb, ROWS_PER_CH)],
                        dsti.at[m % 2])

    def _fire_gather(m):
        return [
            pltpu.async_copy(h_hbm.at[dsti.at[m % 2, k]],
                             hd.at[m % 2, pl.ds(k * SUB, SUB)], gsem)
            for k in range(KSUB)
        ]

    def _fire_scatter(m):
        return [
            pltpu.async_copy(wv.at[m % 2, pl.ds(k * SUB, SUB)],
                             acc_sp.at[srci.at[m % 3, k]], ssem, add=True)
            for k in range(KSUB)
        ]

    def _compute(m):
        b = m % 2

        def _grp(j, carry):
            off = j * 16
            r = j // 8
            q = (j % 8) * 16
            si = srci[m % 3, r, pl.ds(q, 16)]
            di = dsti[b, r, pl.ds(q, 16)]
            z = plsc.load_gather(s_v, [si]) + plsc.load_gather(t_v, [di])
            ee = jnp.exp(-jnp.maximum(z, ALPHA * z))
            plsc.addupdate_scatter(rsum_v, [si], ee)
            for l in range(16):
                wv[b, off + l, :] = hd[b, off + l, :] * ee[l]
            return carry

        lax.fori_loop(0, GROUPS, _grp, 0)

    _stage(0)
    gathers = _fire_gather(0)
    scatters = {}
    for m in range(NCH):
        if m >= 2:
            for cp in scatters.pop(m - 2):
                cp.wait()
        if m + 1 < NCH:
            _stage(m + 1)
        for cp in gathers:
            cp.wait()
        if m + 1 < NCH:
            next_gathers = _fire_gather(m + 1)
        _compute(m)
        scatters[m] = _fire_scatter(m)
        if m + 1 < NCH:
            gathers = next_gathers
    for m in (NCH - 2, NCH - 1):
        for cp in scatters.pop(m):
            cp.wait()

    plsc.subcore_barrier()

    @pl.when(sid < 15)
    def _():
        pltpu.sync_copy(acc_sp.at[pl.ds(sid * WR, WR)], wv.at[0, pl.ds(0, WR)])
        pltpu.sync_copy(wv.at[0, pl.ds(0, WR)],
                        outp.at[c, pl.ds(sid * WR, WR)])

    @pl.when(sid == 15)
    def _():
        pltpu.sync_copy(acc_sp.at[pl.ds(9360, 640)], wv.at[0, pl.ds(0, 640)])
        pltpu.sync_copy(wv.at[0, pl.ds(0, 640)], outp.at[c, pl.ds(9360, 640)])

    pltpu.sync_copy(rsum_v.at[pl.ds(0, N)], outr.at[c, sid])


def _combine_body(p0_ref, p1_ref, r_ref, o_ref):
    num = p0_ref[...] + p1_ref[...]
    den = jnp.sum(r_ref[...], axis=0)
    hp = num / den[:, None]
    o_ref[...] = jnp.where(hp > 0, hp, jnp.exp(jnp.minimum(hp, 0.0)) - 1.0)


_combine = pl.pallas_call(
    _combine_body,
    out_shape=jax.ShapeDtypeStruct((N, C), jnp.float32),
)


def kernel(x, edge_index, W, a):
    src = edge_index[0]
    dst = edge_index[1]
    h, sp, tp = _prep(x, W, a)
    pad = EPAD - E
    srcp = jnp.concatenate([src, jnp.full((pad,), N, jnp.int32)])
    dstp = jnp.concatenate([dst, jnp.zeros((pad,), jnp.int32)])
    src2d = srcp.reshape(EPAD // SUB, SUB)
    dst2d = dstp.reshape(EPAD // SUB, SUB)
    outp, outr = _edges(h, sp, tp, src2d, dst2d)
    return _combine(outp[0], outp[1], outr.reshape(NC * NS, N))


# speedup vs baseline: 23.3959x; 1.1028x over previous
import functools

import jax
import jax.numpy as jnp
from jax import lax
from jax.experimental import pallas as pl
from jax.experimental.pallas import tpu as pltpu
from jax.experimental.pallas import tpu_sc as plsc

N = 10000
E = 320000
DIM = 128
C = 16
ALPHA = 0.2

NC = 2
NS = 16
NW = NC * NS
SUB = 128
KSUB = 8
CHUNK = SUB * KSUB
NCH = 10
EPW = CHUNK * NCH
EPAD = EPW * NW
NPAD = N + 16
GROUPS = CHUNK // 16
PARTIAL = E % CHUNK
ZR = 624
WR = 624


def _prep_body(x_ref, w_ref, a_ref, h_ref, s_ref, t_ref):
    h = jnp.dot(x_ref[...], w_ref[...], preferred_element_type=jnp.float32)
    h_ref[...] = h
    a = a_ref[...]
    s_ref[...] = jnp.concatenate(
        [jnp.sum(h * a[0, :C][None, :], axis=1),
         jnp.zeros((NPAD - N,), jnp.float32)])
    t_ref[...] = jnp.concatenate(
        [jnp.sum(h * a[0, C:][None, :], axis=1),
         jnp.zeros((NPAD - N,), jnp.float32)])


_prep = pl.pallas_call(
    _prep_body,
    out_shape=[
        jax.ShapeDtypeStruct((N, C), jnp.float32),
        jax.ShapeDtypeStruct((NPAD,), jnp.float32),
        jax.ShapeDtypeStruct((NPAD,), jnp.float32),
    ],
)


_mesh = plsc.VectorSubcoreMesh(core_axis_name="c", subcore_axis_name="s")


@functools.partial(
    pl.kernel,
    out_type=[
        jax.ShapeDtypeStruct((NC, N, C), jnp.float32),
        jax.ShapeDtypeStruct((NC, NS, N), jnp.float32),
    ],
    mesh=_mesh,
    compiler_params=pltpu.CompilerParams(needs_layout_passes=False,
                                         use_tc_tiling_on_sc=False),
    scratch_types=[
        pltpu.VMEM((NPAD,), jnp.float32),
        pltpu.VMEM((NPAD,), jnp.float32),
        pltpu.VMEM((NPAD,), jnp.float32),
        pltpu.VMEM((3, CHUNK), jnp.int32),
        pltpu.VMEM((2, CHUNK), jnp.int32),
        pltpu.VMEM((2, CHUNK, C), jnp.float32),
        pltpu.VMEM((2, CHUNK, C), jnp.float32),
        pltpu.VMEM_SHARED((NPAD, C), jnp.float32),
        pltpu.SemaphoreType.DMA,
        pltpu.SemaphoreType.DMA,
    ],
)
def _edges(h_hbm, s_hbm, t_hbm, edge_hbm,
           outp, outr, s_v, t_v, rsum_v, srci, dsti, hd, wv,
           acc_sp, gsem, ssem):
    c = lax.axis_index("c")
    sid = lax.axis_index("s")
    wid = c * NS + sid

    pltpu.sync_copy(s_hbm, s_v)
    pltpu.sync_copy(t_hbm, t_v)

    zero16 = jnp.zeros((16,), jnp.float32)

    def _zrow(j, carry):
        wv[0, j, :] = zero16
        rsum_v[pl.ds(j * 16, 16)] = zero16
        return carry

    lax.fori_loop(0, NPAD // 16, _zrow, 0)

    @pl.when(sid < 15)
    def _():
        pltpu.sync_copy(wv.at[0, pl.ds(0, ZR)],
                        acc_sp.at[pl.ds(sid * ZR, ZR)])

    @pl.when(sid == 15)
    def _():
        pltpu.sync_copy(wv.at[0, pl.ds(0, 656)], acc_sp.at[pl.ds(9360, 656)])

    plsc.subcore_barrier()

    full16 = jnp.full((16,), N, jnp.int32)
    zero16i = jnp.zeros((16,), jnp.int32)

    def _stage(m):
        ss, ds_ = m % 3, m % 2
        start = wid * EPW + m * CHUNK

        def _fill(g0):
            def _f(g, carry):
                srci[ss, pl.ds(g * 16, 16)] = full16
                dsti[ds_, pl.ds(g * 16, 16)] = zero16i
                return carry
            lax.fori_loop(g0, CHUNK // 16, _f, 0)

        @pl.when(start + CHUNK <= E)
        def _():
            pltpu.sync_copy(edge_hbm.at[0, pl.ds(start, CHUNK)], srci.at[ss])
            pltpu.sync_copy(edge_hbm.at[1, pl.ds(start, CHUNK)], dsti.at[ds_])

        @pl.when((start < E) & (start + CHUNK > E))
        def _():
            pltpu.sync_copy(edge_hbm.at[0, pl.ds(start, PARTIAL)],
                            srci.at[ss, pl.ds(0, PARTIAL)])
            pltpu.sync_copy(edge_hbm.at[1, pl.ds(start, PARTIAL)],
                            dsti.at[ds_, pl.ds(0, PARTIAL)])
            _fill(PARTIAL // 16)

        @pl.when(start >= E)
        def _():
            _fill(0)

    def _fire_gather(m):
        return [
            pltpu.async_copy(h_hbm.at[dsti.at[m % 2, pl.ds(k * SUB, SUB)]],
                             hd.at[m % 2, pl.ds(k * SUB, SUB)], gsem)
            for k in range(KSUB)
        ]

    def _fire_scatter(m):
        return [
            pltpu.async_copy(wv.at[m % 2, pl.ds(k * SUB, SUB)],
                             acc_sp.at[srci.at[m % 3, pl.ds(k * SUB, SUB)]],
                             ssem, add=True)
            for k in range(KSUB)
        ]

    def _compute(m):
        b = m % 2

        def _grp(j, carry):
            off = j * 16
            si = srci[m % 3, pl.ds(off, 16)]
            di = dsti[b, pl.ds(off, 16)]
            z = plsc.load_gather(s_v, [si]) + plsc.load_gather(t_v, [di])
            ee = jnp.exp(-jnp.maximum(z, ALPHA * z))
            plsc.addupdate_scatter(rsum_v, [si], ee)
            for l in range(16):
                wv[b, off + l, :] = hd[b, off + l, :] * ee[l]
            return carry

        lax.fori_loop(0, GROUPS, _grp, 0)

    _stage(0)
    gathers = _fire_gather(0)
    scatters = {}
    for m in range(NCH):
        if m >= 2:
            for cp in scatters.pop(m - 2):
                cp.wait()
        if m + 1 < NCH:
            _stage(m + 1)
        for cp in gathers:
            cp.wait()
        if m + 1 < NCH:
            next_gathers = _fire_gather(m + 1)
        _compute(m)
        scatters[m] = _fire_scatter(m)
        if m + 1 < NCH:
            gathers = next_gathers
    for m in (NCH - 2, NCH - 1):
        for cp in scatters.pop(m):
            cp.wait()

    plsc.subcore_barrier()

    @pl.when(sid < 15)
    def _():
        pltpu.sync_copy(acc_sp.at[pl.ds(sid * WR, WR)], wv.at[0, pl.ds(0, WR)])
        pltpu.sync_copy(wv.at[0, pl.ds(0, WR)],
                        outp.at[c, pl.ds(sid * WR, WR)])

    @pl.when(sid == 15)
    def _():
        pltpu.sync_copy(acc_sp.at[pl.ds(9360, 640)], wv.at[0, pl.ds(0, 640)])
        pltpu.sync_copy(wv.at[0, pl.ds(0, 640)], outp.at[c, pl.ds(9360, 640)])

    pltpu.sync_copy(rsum_v.at[pl.ds(0, N)], outr.at[c, sid])


def _combine_body(p_ref, r_ref, o_ref):
    p = p_ref[...]
    num = p[0] + p[1]
    den = jnp.sum(r_ref[...], axis=(0, 1))
    hp = num / den[:, None]
    o_ref[...] = jnp.where(hp > 0, hp, jnp.exp(jnp.minimum(hp, 0.0)) - 1.0)


_combine = pl.pallas_call(
    _combine_body,
    out_shape=jax.ShapeDtypeStruct((N, C), jnp.float32),
)


def kernel(x, edge_index, W, a):
    h, sp, tp = _prep(x, W, a)
    outp, outr = _edges(h, sp, tp, edge_index)
    return _combine(outp, outr)


# speedup vs baseline: 23.4483x; 1.0022x over previous
import functools

import jax
import jax.numpy as jnp
from jax import lax
from jax.experimental import pallas as pl
from jax.experimental.pallas import tpu as pltpu
from jax.experimental.pallas import tpu_sc as plsc

N = 10000
E = 320000
DIM = 128
C = 16
ALPHA = 0.2

NC = 2
NS = 16
NW = NC * NS
SUB = 128
KSUB = 8
CHUNK = SUB * KSUB
NCH = 10
EPW = CHUNK * NCH
EPAD = EPW * NW
NPAD = N + 16
GROUPS = CHUNK // 16
PARTIAL = E % CHUNK
ZR = 624
WR = 624


def _prep_body(x_ref, w_ref, a_ref, h_ref, s_ref, t_ref):
    h = jnp.dot(x_ref[...], w_ref[...], preferred_element_type=jnp.float32)
    h_ref[...] = h
    a = a_ref[...]
    s_ref[...] = jnp.concatenate(
        [jnp.sum(h * a[0, :C][None, :], axis=1),
         jnp.zeros((NPAD - N,), jnp.float32)])
    t_ref[...] = jnp.concatenate(
        [jnp.sum(h * a[0, C:][None, :], axis=1),
         jnp.zeros((NPAD - N,), jnp.float32)])


_prep = pl.pallas_call(
    _prep_body,
    out_shape=[
        jax.ShapeDtypeStruct((N, C), jnp.float32),
        jax.ShapeDtypeStruct((NPAD,), jnp.float32),
        jax.ShapeDtypeStruct((NPAD,), jnp.float32),
    ],
)


_mesh = plsc.VectorSubcoreMesh(core_axis_name="c", subcore_axis_name="s")


@functools.partial(
    pl.kernel,
    out_type=[
        jax.ShapeDtypeStruct((NC, N, C), jnp.float32),
        jax.ShapeDtypeStruct((NC, N), jnp.float32),
    ],
    mesh=_mesh,
    compiler_params=pltpu.CompilerParams(needs_layout_passes=False,
                                         use_tc_tiling_on_sc=False),
    scratch_types=[
        pltpu.VMEM((NPAD,), jnp.float32),
        pltpu.VMEM((NPAD,), jnp.float32),
        pltpu.VMEM((640,), jnp.float32),
        pltpu.VMEM((3, CHUNK), jnp.int32),
        pltpu.VMEM((2, CHUNK), jnp.int32),
        pltpu.VMEM((2, CHUNK, C), jnp.float32),
        pltpu.VMEM((2, CHUNK, C), jnp.float32),
        pltpu.VMEM((2, CHUNK), jnp.float32),
        pltpu.VMEM_SHARED((NPAD, C), jnp.float32),
        pltpu.VMEM_SHARED((NPAD,), jnp.float32),
        pltpu.SemaphoreType.DMA,
        pltpu.SemaphoreType.DMA,
    ],
)
def _edges(h_hbm, s_hbm, t_hbm, edge_hbm,
           outp, outr, s_v, t_v, zbuf, srci, dsti, hd, wv, eev,
           acc_sp, racc_sp, gsem, ssem):
    c = lax.axis_index("c")
    sid = lax.axis_index("s")
    wid = c * NS + sid

    pltpu.sync_copy(s_hbm, s_v)
    pltpu.sync_copy(t_hbm, t_v)

    zero16 = jnp.zeros((16,), jnp.float32)

    def _zrow(j, carry):
        wv[0, j, :] = zero16
        return carry

    lax.fori_loop(0, 656, _zrow, 0)

    def _zbufrow(j, carry):
        zbuf[pl.ds(j * 16, 16)] = zero16
        return carry

    lax.fori_loop(0, 40, _zbufrow, 0)

    @pl.when(sid < 15)
    def _():
        pltpu.sync_copy(wv.at[0, pl.ds(0, ZR)],
                        acc_sp.at[pl.ds(sid * ZR, ZR)])
        pltpu.sync_copy(zbuf, racc_sp.at[pl.ds(sid * 640, 640)])

    @pl.when(sid == 15)
    def _():
        pltpu.sync_copy(wv.at[0, pl.ds(0, 656)], acc_sp.at[pl.ds(9360, 656)])
        pltpu.sync_copy(zbuf.at[pl.ds(0, 416)], racc_sp.at[pl.ds(9600, 416)])

    plsc.subcore_barrier()

    full16 = jnp.full((16,), N, jnp.int32)
    zero16i = jnp.zeros((16,), jnp.int32)

    def _stage(m):
        ss, ds_ = m % 3, m % 2
        start = wid * EPW + m * CHUNK

        def _fill(g0):
            def _f(g, carry):
                srci[ss, pl.ds(g * 16, 16)] = full16
                dsti[ds_, pl.ds(g * 16, 16)] = zero16i
                return carry
            lax.fori_loop(g0, CHUNK // 16, _f, 0)

        @pl.when(start + CHUNK <= E)
        def _():
            pltpu.sync_copy(edge_hbm.at[0, pl.ds(start, CHUNK)], srci.at[ss])
            pltpu.sync_copy(edge_hbm.at[1, pl.ds(start, CHUNK)], dsti.at[ds_])

        @pl.when((start < E) & (start + CHUNK > E))
        def _():
            pltpu.sync_copy(edge_hbm.at[0, pl.ds(start, PARTIAL)],
                            srci.at[ss, pl.ds(0, PARTIAL)])
            pltpu.sync_copy(edge_hbm.at[1, pl.ds(start, PARTIAL)],
                            dsti.at[ds_, pl.ds(0, PARTIAL)])
            _fill(PARTIAL // 16)

        @pl.when(start >= E)
        def _():
            _fill(0)

    def _fire_gather(m):
        return [
            pltpu.async_copy(h_hbm.at[dsti.at[m % 2, pl.ds(k * SUB, SUB)]],
                             hd.at[m % 2, pl.ds(k * SUB, SUB)], gsem)
            for k in range(KSUB)
        ]

    def _fire_scatter(m):
        cps = []
        for k in range(KSUB):
            idx = srci.at[m % 3, pl.ds(k * SUB, SUB)]
            cps.append(pltpu.async_copy(
                wv.at[m % 2, pl.ds(k * SUB, SUB)], acc_sp.at[idx],
                ssem, add=True))
            cps.append(pltpu.async_copy(
                eev.at[m % 2, pl.ds(k * SUB, SUB)], racc_sp.at[idx],
                ssem, add=True))
        return cps

    def _compute(m):
        b = m % 2

        def _grp(j, carry):
            off = j * 16
            si = srci[m % 3, pl.ds(off, 16)]
            di = dsti[b, pl.ds(off, 16)]
            z = plsc.load_gather(s_v, [si]) + plsc.load_gather(t_v, [di])
            ee = jnp.exp(-jnp.maximum(z, ALPHA * z))
            eev[b, pl.ds(off, 16)] = ee
            for l in range(16):
                wv[b, off + l, :] = hd[b, off + l, :] * ee[l]
            return carry

        lax.fori_loop(0, GROUPS, _grp, 0)

    _stage(0)
    gathers = _fire_gather(0)
    scatters = {}
    for m in range(NCH):
        if m >= 2:
            for cp in scatters.pop(m - 2):
                cp.wait()
        if m + 1 < NCH:
            _stage(m + 1)
        for cp in gathers:
            cp.wait()
        if m + 1 < NCH:
            next_gathers = _fire_gather(m + 1)
        _compute(m)
        scatters[m] = _fire_scatter(m)
        if m + 1 < NCH:
            gathers = next_gathers
    for m in (NCH - 2, NCH - 1):
        for cp in scatters.pop(m):
            cp.wait()

    plsc.subcore_barrier()

    @pl.when(sid < 15)
    def _():
        pltpu.sync_copy(acc_sp.at[pl.ds(sid * WR, WR)], wv.at[0, pl.ds(0, WR)])
        pltpu.sync_copy(wv.at[0, pl.ds(0, WR)],
                        outp.at[c, pl.ds(sid * WR, WR)])

    @pl.when(sid == 15)
    def _():
        pltpu.sync_copy(acc_sp.at[pl.ds(9360, 640)], wv.at[0, pl.ds(0, 640)])
        pltpu.sync_copy(wv.at[0, pl.ds(0, 640)], outp.at[c, pl.ds(9360, 640)])

    @pl.when(sid < 15)
    def _():
        pltpu.sync_copy(racc_sp.at[pl.ds(sid * 640, 640)], zbuf)
        pltpu.sync_copy(zbuf, outr.at[c, pl.ds(sid * 640, 640)])

    @pl.when(sid == 15)
    def _():
        pltpu.sync_copy(racc_sp.at[pl.ds(9600, 400)], zbuf.at[pl.ds(0, 400)])
        pltpu.sync_copy(zbuf.at[pl.ds(0, 400)], outr.at[c, pl.ds(9600, 400)])


def _combine_body(p_ref, r_ref, o_ref):
    p = p_ref[...]
    num = p[0] + p[1]
    r = r_ref[...]
    den = r[0] + r[1]
    hp = num / den[:, None]
    o_ref[...] = jnp.where(hp > 0, hp, jnp.exp(jnp.minimum(hp, 0.0)) - 1.0)


_combine = pl.pallas_call(
    _combine_body,
    out_shape=jax.ShapeDtypeStruct((N, C), jnp.float32),
)


def kernel(x, edge_index, W, a):
    h, sp, tp = _prep(x, W, a)
    outp, outr = _edges(h, sp, tp, edge_index)
    return _combine(outp, outr)


# speedup vs baseline: 24.9113x; 1.0624x over previous
import functools

import jax
import jax.numpy as jnp
from jax import lax
from jax.experimental import pallas as pl
from jax.experimental.pallas import tpu as pltpu
from jax.experimental.pallas import tpu_sc as plsc

N = 10000
E = 320000
DIM = 128
C = 16
ALPHA = 0.2

NC = 2
NS = 16
NW = NC * NS
SUB = 128
KSUB = 8
CHUNK = SUB * KSUB
NCH0 = 11
NCH1 = 9
NCHMAX = NCH0
E0 = NCH0 * CHUNK * NS
NPAD = N + 16
GROUPS = CHUNK // 16
PARTIAL = E % CHUNK
ZR = 624
WR = 624


def _prep_body(x_ref, w_ref, a_ref, h_ref, s_ref, t_ref):
    h = jnp.dot(x_ref[...], w_ref[...], preferred_element_type=jnp.float32)
    h_ref[...] = h
    a = a_ref[...]
    s_ref[...] = jnp.concatenate(
        [jnp.sum(h * a[0, :C][None, :], axis=1),
         jnp.zeros((NPAD - N,), jnp.float32)])
    t_ref[...] = jnp.concatenate(
        [jnp.sum(h * a[0, C:][None, :], axis=1),
         jnp.zeros((NPAD - N,), jnp.float32)])


_prep = pl.pallas_call(
    _prep_body,
    out_shape=[
        jax.ShapeDtypeStruct((N, C), jnp.float32),
        jax.ShapeDtypeStruct((NPAD,), jnp.float32),
        jax.ShapeDtypeStruct((NPAD,), jnp.float32),
    ],
)


_mesh = plsc.VectorSubcoreMesh(core_axis_name="c", subcore_axis_name="s")


@functools.partial(
    pl.kernel,
    out_type=[
        jax.ShapeDtypeStruct((NC, N, C), jnp.float32),
        jax.ShapeDtypeStruct((NC, N), jnp.float32),
    ],
    mesh=_mesh,
    compiler_params=pltpu.CompilerParams(needs_layout_passes=False,
                                         use_tc_tiling_on_sc=False),
    scratch_types=[
        pltpu.VMEM((NPAD,), jnp.float32),
        pltpu.VMEM((NPAD,), jnp.float32),
        pltpu.VMEM((640,), jnp.float32),
        pltpu.VMEM((3, CHUNK), jnp.int32),
        pltpu.VMEM((2, CHUNK), jnp.int32),
        pltpu.VMEM((2, CHUNK, C), jnp.float32),
        pltpu.VMEM((2, CHUNK, C), jnp.float32),
        pltpu.VMEM((2, CHUNK), jnp.float32),
        pltpu.VMEM_SHARED((NPAD, C), jnp.float32),
        pltpu.VMEM_SHARED((NPAD,), jnp.float32),
        pltpu.SemaphoreType.DMA,
        pltpu.SemaphoreType.DMA,
    ],
)
def _edges(h_hbm, s_hbm, t_hbm, edge_hbm,
           outp, outr, s_v, t_v, zbuf, srci, dsti, hd, wv, eev,
           acc_sp, racc_sp, gsem, ssem):
    c = lax.axis_index("c")
    sid = lax.axis_index("s")
    wid = c * NS + sid

    pltpu.sync_copy(s_hbm, s_v)
    pltpu.sync_copy(t_hbm, t_v)

    zero16 = jnp.zeros((16,), jnp.float32)

    def _zrow(j, carry):
        wv[0, j, :] = zero16
        return carry

    lax.fori_loop(0, 656, _zrow, 0)

    def _zbufrow(j, carry):
        zbuf[pl.ds(j * 16, 16)] = zero16
        return carry

    lax.fori_loop(0, 40, _zbufrow, 0)

    @pl.when(sid < 15)
    def _():
        pltpu.sync_copy(wv.at[0, pl.ds(0, ZR)],
                        acc_sp.at[pl.ds(sid * ZR, ZR)])
        pltpu.sync_copy(zbuf, racc_sp.at[pl.ds(sid * 640, 640)])

    @pl.when(sid == 15)
    def _():
        pltpu.sync_copy(wv.at[0, pl.ds(0, 656)], acc_sp.at[pl.ds(9360, 656)])
        pltpu.sync_copy(zbuf.at[pl.ds(0, 416)], racc_sp.at[pl.ds(9600, 416)])

    plsc.subcore_barrier()

    full16 = jnp.full((16,), N, jnp.int32)
    zero16i = jnp.zeros((16,), jnp.int32)

    nch_c = jnp.where(c == 0, NCH0, NCH1)
    wbase = jnp.where(c == 0, sid * (NCH0 * CHUNK), E0 + sid * (NCH1 * CHUNK))

    def _start(m):
        return wbase + m * CHUNK

    def _pred(m):
        return (m < nch_c) & (_start(m) < E)

    def _stage(m):
        ss, ds_ = m % 3, m % 2
        start = _start(m)

        def _fill(g0):
            def _f(g, carry):
                srci[ss, pl.ds(g * 16, 16)] = full16
                dsti[ds_, pl.ds(g * 16, 16)] = zero16i
                return carry
            lax.fori_loop(g0, CHUNK // 16, _f, 0)

        @pl.when(start + CHUNK <= E)
        def _():
            pltpu.sync_copy(edge_hbm.at[0, pl.ds(start, CHUNK)], srci.at[ss])
            pltpu.sync_copy(edge_hbm.at[1, pl.ds(start, CHUNK)], dsti.at[ds_])

        @pl.when((start < E) & (start + CHUNK > E))
        def _():
            pltpu.sync_copy(edge_hbm.at[0, pl.ds(start, PARTIAL)],
                            srci.at[ss, pl.ds(0, PARTIAL)])
            pltpu.sync_copy(edge_hbm.at[1, pl.ds(start, PARTIAL)],
                            dsti.at[ds_, pl.ds(0, PARTIAL)])
            _fill(PARTIAL // 16)


    def _fire_gather(m):
        for k in range(KSUB):
            pltpu.async_copy(h_hbm.at[dsti.at[m % 2, pl.ds(k * SUB, SUB)]],
                             hd.at[m % 2, pl.ds(k * SUB, SUB)], gsem)

    def _fire_scatter(m):
        for k in range(KSUB):
            idx = srci.at[m % 3, pl.ds(k * SUB, SUB)]
            pltpu.async_copy(wv.at[m % 2, pl.ds(k * SUB, SUB)],
                             acc_sp.at[idx], ssem, add=True)
            pltpu.async_copy(eev.at[m % 2, pl.ds(k * SUB, SUB)],
                             racc_sp.at[idx], ssem, add=True)

    def _compute(m):
        b = m % 2

        def _grp(j, carry):
            off = j * 16
            si = srci[m % 3, pl.ds(off, 16)]
            di = dsti[b, pl.ds(off, 16)]
            z = plsc.load_gather(s_v, [si]) + plsc.load_gather(t_v, [di])
            ee = jnp.exp(-jnp.maximum(z, ALPHA * z))
            eev[b, pl.ds(off, 16)] = ee
            for l in range(16):
                wv[b, off + l, :] = hd[b, off + l, :] * ee[l]
            return carry

        lax.fori_loop(0, GROUPS, _grp, 0)

    def _pstage(m):
        @pl.when(_pred(m))
        def _():
            _stage(m)

    def _pfire_gather(m):
        @pl.when(_pred(m))
        def _():
            _fire_gather(m)

    def _pcompute(m):
        @pl.when(_pred(m))
        def _():
            _compute(m)

    def _pfire_scatter(m):
        @pl.when(_pred(m))
        def _():
            _fire_scatter(m)

    def _wait_gather(m):
        @pl.when(_pred(m))
        def _():
            for k in range(KSUB):
                pltpu.make_async_copy(
                    h_hbm.at[dsti.at[m % 2, pl.ds(k * SUB, SUB)]],
                    hd.at[m % 2, pl.ds(k * SUB, SUB)], gsem).wait()

    def _wait_scatter(m):
        @pl.when(_pred(m))
        def _():
            for k in range(KSUB):
                idx = srci.at[m % 3, pl.ds(k * SUB, SUB)]
                pltpu.make_async_copy(
                    wv.at[m % 2, pl.ds(k * SUB, SUB)], acc_sp.at[idx],
                    ssem).wait()
                pltpu.make_async_copy(
                    eev.at[m % 2, pl.ds(k * SUB, SUB)], racc_sp.at[idx],
                    ssem).wait()

    _pstage(0)
    _pfire_gather(0)
    for m in range(NCHMAX):
        if m >= 2:
            _wait_scatter(m - 2)
        if m + 1 < NCHMAX:
            _pstage(m + 1)
        _wait_gather(m)
        if m + 1 < NCHMAX:
            _pfire_gather(m + 1)
        _pcompute(m)
        _pfire_scatter(m)
    for m in (NCHMAX - 2, NCHMAX - 1):
        _wait_scatter(m)

    plsc.subcore_barrier()

    @pl.when(sid < 15)
    def _():
        pltpu.sync_copy(acc_sp.at[pl.ds(sid * WR, WR)], wv.at[0, pl.ds(0, WR)])
        pltpu.sync_copy(wv.at[0, pl.ds(0, WR)],
                        outp.at[c, pl.ds(sid * WR, WR)])

    @pl.when(sid == 15)
    def _():
        pltpu.sync_copy(acc_sp.at[pl.ds(9360, 640)], wv.at[0, pl.ds(0, 640)])
        pltpu.sync_copy(wv.at[0, pl.ds(0, 640)], outp.at[c, pl.ds(9360, 640)])

    @pl.when(sid < 15)
    def _():
        pltpu.sync_copy(racc_sp.at[pl.ds(sid * 640, 640)], zbuf)
        pltpu.sync_copy(zbuf, outr.at[c, pl.ds(sid * 640, 640)])

    @pl.when(sid == 15)
    def _():
        pltpu.sync_copy(racc_sp.at[pl.ds(9600, 400)], zbuf.at[pl.ds(0, 400)])
        pltpu.sync_copy(zbuf.at[pl.ds(0, 400)], outr.at[c, pl.ds(9600, 400)])


def _combine_body(p_ref, r_ref, o_ref):
    p = p_ref[...]
    num = p[0] + p[1]
    r = r_ref[...]
    den = r[0] + r[1]
    hp = num / den[:, None]
    o_ref[...] = jnp.where(hp > 0, hp, jnp.exp(jnp.minimum(hp, 0.0)) - 1.0)


_combine = pl.pallas_call(
    _combine_body,
    out_shape=jax.ShapeDtypeStruct((N, C), jnp.float32),
)


def kernel(x, edge_index, W, a):
    h, sp, tp = _prep(x, W, a)
    outp, outr = _edges(h, sp, tp, edge_index)
    return _combine(outp, outr)


# speedup vs baseline: 26.3751x; 1.0588x over previous
import functools

import jax
import jax.numpy as jnp
from jax import lax
from jax.experimental import pallas as pl
from jax.experimental.pallas import tpu as pltpu
from jax.experimental.pallas import tpu_sc as plsc

N = 10000
E = 320000
DIM = 128
C = 16
ALPHA = 0.2

NC = 2
NS = 16
NW = NC * NS
SUB = 128
KSUB = 8
CHUNK = SUB * KSUB
NCH0 = 10
NCH1 = 10
NCHMAX = NCH0
E0 = NCH0 * CHUNK * NS
NPAD = N + 16
GROUPS = CHUNK // 16
PARTIAL = E % CHUNK
ZR = 624
WR = 624


def _prep_body(x_ref, w_ref, a_ref, h_ref, s_ref, t_ref):
    h = jnp.dot(x_ref[...], w_ref[...], preferred_element_type=jnp.float32)
    h_ref[...] = h
    a = a_ref[...]
    s_ref[...] = jnp.concatenate(
        [jnp.sum(h * a[0, :C][None, :], axis=1),
         jnp.zeros((NPAD - N,), jnp.float32)])
    t_ref[...] = jnp.concatenate(
        [jnp.sum(h * a[0, C:][None, :], axis=1),
         jnp.zeros((NPAD - N,), jnp.float32)])


_prep = pl.pallas_call(
    _prep_body,
    out_shape=[
        jax.ShapeDtypeStruct((N, C), jnp.float32),
        jax.ShapeDtypeStruct((NPAD,), jnp.float32),
        jax.ShapeDtypeStruct((NPAD,), jnp.float32),
    ],
)


_mesh = plsc.VectorSubcoreMesh(core_axis_name="c", subcore_axis_name="s")


@functools.partial(
    pl.kernel,
    out_type=[
        jax.ShapeDtypeStruct((NC, N, C), jnp.float32),
        jax.ShapeDtypeStruct((NC, N), jnp.float32),
    ],
    mesh=_mesh,
    compiler_params=pltpu.CompilerParams(needs_layout_passes=False,
                                         use_tc_tiling_on_sc=False),
    scratch_types=[
        pltpu.VMEM((NPAD,), jnp.float32),
        pltpu.VMEM((NPAD,), jnp.float32),
        pltpu.VMEM((640,), jnp.float32),
        pltpu.VMEM((3, CHUNK), jnp.int32),
        pltpu.VMEM((2, CHUNK), jnp.int32),
        pltpu.VMEM((2, CHUNK, C), jnp.float32),
        pltpu.VMEM((2, CHUNK, C), jnp.float32),
        pltpu.VMEM((2, CHUNK), jnp.float32),
        pltpu.VMEM_SHARED((NPAD, C), jnp.float32),
        pltpu.VMEM_SHARED((NPAD,), jnp.float32),
        pltpu.SemaphoreType.DMA,
        pltpu.SemaphoreType.DMA,
    ],
)
def _edges(h_hbm, s_hbm, t_hbm, edge_hbm,
           outp, outr, s_v, t_v, zbuf, srci, dsti, hd, wv, eev,
           acc_sp, racc_sp, gsem, ssem):
    c = lax.axis_index("c")
    sid = lax.axis_index("s")
    wid = c * NS + sid

    pltpu.sync_copy(s_hbm, s_v)
    pltpu.sync_copy(t_hbm, t_v)

    zero16 = jnp.zeros((16,), jnp.float32)

    def _zrow(j, carry):
        wv[0, j, :] = zero16
        return carry

    lax.fori_loop(0, 656, _zrow, 0)

    def _zbufrow(j, carry):
        zbuf[pl.ds(j * 16, 16)] = zero16
        return carry

    lax.fori_loop(0, 40, _zbufrow, 0)

    @pl.when(sid < 15)
    def _():
        pltpu.sync_copy(wv.at[0, pl.ds(0, ZR)],
                        acc_sp.at[pl.ds(sid * ZR, ZR)])
        pltpu.sync_copy(zbuf, racc_sp.at[pl.ds(sid * 640, 640)])

    @pl.when(sid == 15)
    def _():
        pltpu.sync_copy(wv.at[0, pl.ds(0, 656)], acc_sp.at[pl.ds(9360, 656)])
        pltpu.sync_copy(zbuf.at[pl.ds(0, 416)], racc_sp.at[pl.ds(9600, 416)])

    plsc.subcore_barrier()

    full16 = jnp.full((16,), N, jnp.int32)
    zero16i = jnp.zeros((16,), jnp.int32)

    nch_c = jnp.where(c == 0, NCH0, NCH1)
    wbase = jnp.where(c == 0, sid * (NCH0 * CHUNK), E0 + sid * (NCH1 * CHUNK))

    def _start(m):
        return wbase + m * CHUNK

    def _pred(m):
        return (m < nch_c) & (_start(m) < E)

    def _stage(m):
        ss, ds_ = m % 3, m % 2
        start = _start(m)

        def _fill(g0):
            def _f(g, carry):
                srci[ss, pl.ds(g * 16, 16)] = full16
                dsti[ds_, pl.ds(g * 16, 16)] = zero16i
                return carry
            lax.fori_loop(g0, CHUNK // 16, _f, 0)

        @pl.when(start + CHUNK <= E)
        def _():
            pltpu.sync_copy(edge_hbm.at[0, pl.ds(start, CHUNK)], srci.at[ss])
            pltpu.sync_copy(edge_hbm.at[1, pl.ds(start, CHUNK)], dsti.at[ds_])

        @pl.when((start < E) & (start + CHUNK > E))
        def _():
            pltpu.sync_copy(edge_hbm.at[0, pl.ds(start, PARTIAL)],
                            srci.at[ss, pl.ds(0, PARTIAL)])
            pltpu.sync_copy(edge_hbm.at[1, pl.ds(start, PARTIAL)],
                            dsti.at[ds_, pl.ds(0, PARTIAL)])
            _fill(PARTIAL // 16)


    def _fire_gather(m):
        for k in range(KSUB):
            pltpu.async_copy(h_hbm.at[dsti.at[m % 2, pl.ds(k * SUB, SUB)]],
                             hd.at[m % 2, pl.ds(k * SUB, SUB)], gsem)

    def _fire_scatter(m):
        for k in range(KSUB):
            idx = srci.at[m % 3, pl.ds(k * SUB, SUB)]
            pltpu.async_copy(wv.at[m % 2, pl.ds(k * SUB, SUB)],
                             acc_sp.at[idx], ssem, add=True)
            pltpu.async_copy(eev.at[m % 2, pl.ds(k * SUB, SUB)],
                             racc_sp.at[idx], ssem, add=True)

    def _compute(m):
        b = m % 2

        def _grp(j, carry):
            off = j * 16
            si = srci[m % 3, pl.ds(off, 16)]
            di = dsti[b, pl.ds(off, 16)]
            z = plsc.load_gather(s_v, [si]) + plsc.load_gather(t_v, [di])
            ee = jnp.exp(-jnp.maximum(z, ALPHA * z))
            eev[b, pl.ds(off, 16)] = ee
            for l in range(16):
                wv[b, off + l, :] = hd[b, off + l, :] * ee[l]
            return carry

        lax.fori_loop(0, GROUPS, _grp, 0)

    def _pstage(m):
        @pl.when(_pred(m))
        def _():
            _stage(m)

    def _pfire_gather(m):
        @pl.when(_pred(m))
        def _():
            _fire_gather(m)

    def _pcompute(m):
        @pl.when(_pred(m))
        def _():
            _compute(m)

    def _pfire_scatter(m):
        @pl.when(_pred(m))
        def _():
            _fire_scatter(m)

    def _wait_gather(m):
        @pl.when(_pred(m))
        def _():
            for k in range(KSUB):
                pltpu.make_async_copy(
                    h_hbm.at[dsti.at[m % 2, pl.ds(k * SUB, SUB)]],
                    hd.at[m % 2, pl.ds(k * SUB, SUB)], gsem).wait()

    def _wait_scatter(m):
        @pl.when(_pred(m))
        def _():
            for k in range(KSUB):
                idx = srci.at[m % 3, pl.ds(k * SUB, SUB)]
                pltpu.make_async_copy(
                    wv.at[m % 2, pl.ds(k * SUB, SUB)], acc_sp.at[idx],
                    ssem).wait()
                pltpu.make_async_copy(
                    eev.at[m % 2, pl.ds(k * SUB, SUB)], racc_sp.at[idx],
                    ssem).wait()

    _pstage(0)
    _pfire_gather(0)
    for m in range(NCHMAX):
        if m >= 2:
            _wait_scatter(m - 2)
        if m + 1 < NCHMAX:
            _pstage(m + 1)
        _wait_gather(m)
        if m + 1 < NCHMAX:
            _pfire_gather(m + 1)
        _pcompute(m)
        _pfire_scatter(m)
    for m in (NCHMAX - 2, NCHMAX - 1):
        _wait_scatter(m)

    plsc.subcore_barrier()

    @pl.when(sid < 15)
    def _():
        pltpu.sync_copy(acc_sp.at[pl.ds(sid * WR, WR)], wv.at[0, pl.ds(0, WR)])
        pltpu.sync_copy(wv.at[0, pl.ds(0, WR)],
                        outp.at[c, pl.ds(sid * WR, WR)])

    @pl.when(sid == 15)
    def _():
        pltpu.sync_copy(acc_sp.at[pl.ds(9360, 640)], wv.at[0, pl.ds(0, 640)])
        pltpu.sync_copy(wv.at[0, pl.ds(0, 640)], outp.at[c, pl.ds(9360, 640)])

    @pl.when(sid < 15)
    def _():
        pltpu.sync_copy(racc_sp.at[pl.ds(sid * 640, 640)], zbuf)
        pltpu.sync_copy(zbuf, outr.at[c, pl.ds(sid * 640, 640)])

    @pl.when(sid == 15)
    def _():
        pltpu.sync_copy(racc_sp.at[pl.ds(9600, 400)], zbuf.at[pl.ds(0, 400)])
        pltpu.sync_copy(zbuf.at[pl.ds(0, 400)], outr.at[c, pl.ds(9600, 400)])


def _combine_body(p_ref, r_ref, o_ref):
    p = p_ref[...]
    num = p[0] + p[1]
    r = r_ref[...]
    den = r[0] + r[1]
    hp = num / den[:, None]
    o_ref[...] = jnp.where(hp > 0, hp, jnp.exp(jnp.minimum(hp, 0.0)) - 1.0)


_combine = pl.pallas_call(
    _combine_body,
    out_shape=jax.ShapeDtypeStruct((N, C), jnp.float32),
)


def kernel(x, edge_index, W, a):
    h, sp, tp = _prep(x, W, a)
    outp, outr = _edges(h, sp, tp, edge_index)
    return _combine(outp, outr)
